# Initial kernel scaffold; baseline (speedup 1.0000x reference)
import jax
import jax.numpy as jnp
from jax.experimental import pallas as pl


def kernel(n_feats, edge_index, W1, b1, g1, be1, W2, b2, Wa, ba, gp, bp, Wb, bb):
    raise NotImplementedError("write your pallas kernel here")


# speedup vs baseline: 5.9635x; 5.9635x over previous
import functools

import jax
import jax.numpy as jnp
from jax import lax
from jax.experimental import pallas as pl
from jax.experimental.pallas import tpu as pltpu
from jax.experimental.pallas import tpu_sc as plsc

NC = 2
NS = 16
NW = NC * NS
LN = 16


def _pick_block(epw, mult=1, cap=10**9, even=True):
    for b in (512, 400, 320, 256, 200, 160, 144, 128, 120, 112, 100, 96, 80,
              64, 50, 48, 40, 32, 25, 20, 16, 10, 8, 5, 4, 2, 1):
        if b > cap or b % mult != 0:
            continue
        if epw % b == 0 and (not even or (epw // b) % 2 == 0):
            return b
    return 1


def _sc_mesh():
    return plsc.VectorSubcoreMesh(core_axis_name="c", subcore_axis_name="s")


def _stripe(n_pad):
    rpt = n_pad // NS
    for z in (80, 64, 32, 16, 8, 4, 2, 1):
        if rpt % z == 0:
            return rpt, z
    return rpt, 1


def _make_hist(n_pad, nblk, b):
    rpt, _ = _stripe(n_pad)
    zlen = (rpt + LN - 1) // LN * LN

    @functools.partial(
        pl.kernel,
        out_type=jax.ShapeDtypeStruct((NC, 2, NS, rpt), jnp.float32),
        mesh=_sc_mesh(),
        scratch_types=[
            pltpu.VMEM_SHARED((n_pad,), jnp.float32),
            pltpu.VMEM_SHARED((n_pad,), jnp.float32),
            pltpu.VMEM((nblk * b,), jnp.int32),
            pltpu.VMEM((nblk * b,), jnp.int32),
            pltpu.VMEM((b,), jnp.float32),
            pltpu.VMEM((zlen,), jnp.float32),
        ],
    )
    def hist_k(srcr, dstr, out, acc_s, acc_d, idx_s, idx_d, ones_v, zb):
        c = lax.axis_index("c")
        s = lax.axis_index("s")
        wid = s * NC + c
        r0 = s * rpt

        def fill_zb(j, carry):
            zb[pl.ds(j * LN, LN)] = jnp.zeros((LN,), jnp.float32)
            return carry

        lax.fori_loop(0, zlen // LN, fill_zb, 0)

        def fill_ones(j, carry):
            ones_v[pl.ds(j * LN, LN)] = jnp.ones((LN,), jnp.float32)
            return carry

        lax.fori_loop(0, b // LN, fill_ones, 0)

        pltpu.sync_copy(zb.at[pl.ds(0, rpt)], acc_s.at[pl.ds(r0, rpt)])
        pltpu.sync_copy(zb.at[pl.ds(0, rpt)], acc_d.at[pl.ds(r0, rpt)])
        epw = nblk * b
        pltpu.sync_copy(srcr.at[pl.ds(wid * epw, epw)], idx_s)
        pltpu.sync_copy(dstr.at[pl.ds(wid * epw, epw)], idx_d)
        plsc.subcore_barrier()

        def step(g, carry):
            pltpu.sync_copy(ones_v, acc_s.at[idx_s.at[pl.ds(g * b, b)]], add=True)
            pltpu.sync_copy(ones_v, acc_d.at[idx_d.at[pl.ds(g * b, b)]], add=True)
            return carry

        lax.fori_loop(0, nblk, step, 0)
        plsc.subcore_barrier()
        pltpu.sync_copy(acc_s.at[pl.ds(r0, rpt)], out.at[c, 0, s])
        pltpu.sync_copy(acc_d.at[pl.ds(r0, rpt)], out.at[c, 1, s])

    return hist_k


def _make_agg(n_pad, f, epw, rounds, nblk, b):
    rpt, zrf = _stripe(n_pad)
    spc = epw // rounds

    @functools.partial(
        pl.kernel,
        out_type=jax.ShapeDtypeStruct((NC, n_pad, f), jnp.float32),
        mesh=_sc_mesh(),
        scratch_types=[
            pltpu.VMEM_SHARED((n_pad, f), jnp.float32),
            pltpu.VMEM((spc,), jnp.int32),
            pltpu.VMEM((spc,), jnp.int32),
            pltpu.VMEM((2, b, f), jnp.float32),
            pltpu.VMEM((zrf, f), jnp.float32),
            pltpu.SemaphoreType.DMA((2,)),
        ],
    )
    def agg_k(h, srcr, dstr, out, acc, idx_s, idx_d, buf, zb, sem):
        c = lax.axis_index("c")
        s = lax.axis_index("s")
        wid = s * NC + c
        r0 = s * rpt

        def fill_zb(j, carry):
            for k in range(f // LN):
                zb[j, pl.ds(k * LN, LN)] = jnp.zeros((LN,), jnp.float32)
            return carry

        lax.fori_loop(0, zrf, fill_zb, 0)

        def zero_acc(i, carry):
            pltpu.sync_copy(zb, acc.at[pl.ds(r0 + i * zrf, zrf)])
            return carry

        lax.fori_loop(0, rpt // zrf, zero_acc, 0)
        plsc.subcore_barrier()

        def round_body(r, carry):
            e0 = wid * epw + r * spc
            pltpu.sync_copy(srcr.at[pl.ds(e0, spc)], idx_s)
            pltpu.sync_copy(dstr.at[pl.ds(e0, spc)], idx_d)
            pltpu.async_copy(h.at[idx_s.at[pl.ds(0, b)]], buf.at[0], sem.at[0])

            def step2(g2, carry2):
                g0 = g2 * 2
                for par in range(2):
                    g = g0 + par
                    nxt = g + 1

                    @pl.when(nxt < nblk)
                    def _(nxt=nxt, par=par):
                        pltpu.async_copy(
                            h.at[idx_s.at[pl.ds(nxt * b, b)]],
                            buf.at[1 - par], sem.at[1 - par]
                        )

                    pltpu.make_async_copy(
                        h.at[idx_s.at[pl.ds(g * b, b)]], buf.at[par], sem.at[par]
                    ).wait()
                    pltpu.sync_copy(
                        buf.at[par], acc.at[idx_d.at[pl.ds(g * b, b)]], add=True
                    )
                return carry2

            lax.fori_loop(0, nblk // 2, step2, 0)
            return carry

        lax.fori_loop(0, rounds, round_body, 0)
        plsc.subcore_barrier()
        pltpu.sync_copy(acc.at[pl.ds(r0, rpt)], out.at[c, pl.ds(r0, rpt)])

    return agg_k


def _make_edge(n, h, e, nblk, b):
    hv = h // LN
    epw = e // NW

    @functools.partial(
        pl.kernel,
        out_type=(
            jax.ShapeDtypeStruct((e, h), jnp.float32),
            jax.ShapeDtypeStruct((NW, 2, h), jnp.float32),
        ),
        mesh=_sc_mesh(),
        scratch_types=[
            pltpu.VMEM((nblk * b,), jnp.int32),
            pltpu.VMEM((nblk * b,), jnp.int32),
            pltpu.VMEM((2, b, h), jnp.float32),
            pltpu.VMEM((2, b, h), jnp.float32),
            pltpu.VMEM((2, b, h), jnp.float32),
            pltpu.VMEM((2, h), jnp.float32),
            pltpu.SemaphoreType.DMA((2,)),
            pltpu.SemaphoreType.DMA((2,)),
            pltpu.SemaphoreType.DMA((2,)),
        ],
    )
    def edge_k(p_h, q_h, srcr, dstr, z_h, parts, idx_s, idx_d, buf_p, buf_q,
               buf_z, sbuf, sem_p, sem_q, sem_z):
        c = lax.axis_index("c")
        s = lax.axis_index("s")
        wid = s * NC + c
        base = wid * epw

        pltpu.sync_copy(srcr.at[pl.ds(base, epw)], idx_s)
        pltpu.sync_copy(dstr.at[pl.ds(base, epw)], idx_d)
        pltpu.async_copy(p_h.at[idx_s.at[pl.ds(0, b)]], buf_p.at[0], sem_p.at[0])
        pltpu.async_copy(q_h.at[idx_d.at[pl.ds(0, b)]], buf_q.at[0], sem_q.at[0])

        zvec = jnp.zeros((LN,), jnp.float32)
        init = (zvec,) * (2 * hv)

        def pair(g2, carry):
            g0 = g2 * 2
            for par in range(2):
                g = g0 + par
                nxt = g + 1

                @pl.when(nxt < nblk)
                def _(nxt=nxt, par=par):
                    pltpu.async_copy(
                        p_h.at[idx_s.at[pl.ds(nxt * b, b)]],
                        buf_p.at[1 - par], sem_p.at[1 - par]
                    )
                    pltpu.async_copy(
                        q_h.at[idx_d.at[pl.ds(nxt * b, b)]],
                        buf_q.at[1 - par], sem_q.at[1 - par]
                    )

                pltpu.make_async_copy(
                    p_h.at[idx_s.at[pl.ds(g * b, b)]], buf_p.at[par], sem_p.at[par]
                ).wait()
                pltpu.make_async_copy(
                    q_h.at[idx_d.at[pl.ds(g * b, b)]], buf_q.at[par], sem_q.at[par]
                ).wait()

                @pl.when(g >= 2)
                def _(g=g, par=par):
                    pltpu.make_async_copy(
                        buf_z.at[par],
                        z_h.at[pl.ds(base + (g - 2) * b, b)],
                        sem_z.at[par],
                    ).wait()

                def row(j, cr, par=par):
                    new_s = []
                    new_t = []
                    for k in range(hv):
                        pv = buf_p[par, j, pl.ds(k * LN, LN)]
                        qv = buf_q[par, j, pl.ds(k * LN, LN)]
                        zv = jnp.maximum(pv + qv, 0.0)
                        buf_z[par, j, pl.ds(k * LN, LN)] = zv
                        new_s.append(cr[k] + zv)
                        new_t.append(cr[hv + k] + zv * zv)
                    return tuple(new_s) + tuple(new_t)

                carry = lax.fori_loop(0, b, row, carry)
                pltpu.async_copy(
                    buf_z.at[par], z_h.at[pl.ds(base + g * b, b)], sem_z.at[par]
                )
            return carry

        carry = lax.fori_loop(0, nblk // 2, pair, init)

        for g in (nblk - 2, nblk - 1):
            pltpu.make_async_copy(
                buf_z.at[g % 2], z_h.at[pl.ds(base + g * b, b)], sem_z.at[g % 2]
            ).wait()

        for k in range(hv):
            sbuf[0, pl.ds(k * LN, LN)] = carry[k]
            sbuf[1, pl.ds(k * LN, LN)] = carry[hv + k]
        pltpu.sync_copy(sbuf, parts.at[wid])

    return edge_k


def kernel(n_feats, edge_index, W1, b1, g1, be1, W2, b2, Wa, ba, gp, bp, Wb, bb):
    n, f = n_feats.shape
    e = edge_index.shape[1]
    h = W1.shape[1]
    cdim = Wb.shape[1]
    epw = e // NW
    rpt8 = ((n + NS - 1) // NS + 127) // 128 * 128
    n_pad = NS * rpt8

    b_a = _pick_block(epw, mult=8, cap=100)
    rounds_a = epw // (2 * b_a)
    for r in range(1, epw + 1):
        if epw % r == 0:
            spc = epw // r
            if spc % b_a == 0 and (spc // b_a) % 2 == 0 and spc <= 2048:
                rounds_a = r
                break
    nblk_a = (epw // rounds_a) // b_a
    b_h = _pick_block(epw, mult=16, even=False)
    nblk_h = epw // b_h
    b_e = _pick_block(epw, mult=8, cap=64)
    nblk_e = epw // b_e

    src_flat = edge_index[0]
    dst_flat = edge_index[1]

    hist_k = _make_hist(n_pad, nblk_h, b_h)
    agg_k = _make_agg(n_pad, h, epw, rounds_a, nblk_a, b_a)
    edge_k = _make_edge(n_pad, h, e, nblk_e, b_e)

    cnt = hist_k(src_flat, dst_flat)
    deg = cnt.sum(axis=0).reshape(2, n_pad)[:, :n]
    deg_s = deg[0].reshape(n, 1)
    deg_d = deg[1].reshape(n, 1)

    def tc1_body(cnt_ref, x_ref, h1_ref):
        h1_ref[...] = x_ref[...] * lax.rsqrt(jnp.maximum(cnt_ref[...], 1.0))

    h1 = pl.pallas_call(
        tc1_body,
        out_shape=jax.ShapeDtypeStruct((n, f), jnp.float32),
    )(deg_s, n_feats)

    agg1 = agg_k(h1, src_flat, dst_flat)

    def tc2_body(agg_ref, ds_ref, dd_ref, w1_ref, b1_ref, g1_ref, be1_ref, h2_ref):
        c_in = dd_ref[...]
        c_out = ds_ref[...]
        agg = (agg_ref[0, 0:n] + agg_ref[1, 0:n]) * lax.rsqrt(jnp.maximum(c_in, 1.0))
        x = jnp.dot(agg, w1_ref[...], preferred_element_type=jnp.float32)
        x = jnp.maximum(x + b1_ref[...], 0.0)
        m = jnp.mean(x, axis=0, keepdims=True)
        v = jnp.mean((x - m) * (x - m), axis=0, keepdims=True)
        xb = (x - m) * lax.rsqrt(v + 1e-5) * g1_ref[...] + be1_ref[...]
        h2_ref[...] = xb * lax.rsqrt(jnp.maximum(c_out, 1.0))

    h2 = pl.pallas_call(
        tc2_body,
        out_shape=jax.ShapeDtypeStruct((n, h), jnp.float32),
    )(agg1, deg_s, deg_d, W1, b1.reshape(1, h), g1.reshape(1, h), be1.reshape(1, h))

    agg2 = agg_k(h2, src_flat, dst_flat)

    def tc3_body(agg_ref, dd_ref, w2_ref, b2_ref, wa_ref, ba_ref, p_ref, q_ref):
        c_in = dd_ref[...]
        agg = (agg_ref[0, 0:n] + agg_ref[1, 0:n]) * lax.rsqrt(jnp.maximum(c_in, 1.0))
        x2 = jnp.dot(agg, w2_ref[...], preferred_element_type=jnp.float32)
        x2 = x2 + b2_ref[...]
        p_ref[...] = (
            jnp.dot(x2, wa_ref[0:h, :], preferred_element_type=jnp.float32)
            + ba_ref[...]
        )
        q_ref[...] = jnp.dot(
            x2, wa_ref[h : 2 * h, :], preferred_element_type=jnp.float32
        )

    p_nodes, q_nodes = pl.pallas_call(
        tc3_body,
        out_shape=(
            jax.ShapeDtypeStruct((n, h), jnp.float32),
            jax.ShapeDtypeStruct((n, h), jnp.float32),
        ),
    )(agg2, deg_d, W2, b2.reshape(1, h), Wa, ba.reshape(1, h))

    z_edges, parts = edge_k(p_nodes, q_nodes, src_flat, dst_flat)

    be_blk = 3200
    while e % be_blk != 0:
        be_blk //= 2
    grid = (e // be_blk,)

    def tc4_body(part_ref, gp_ref, bp_ref, wb_ref, bb_ref, z_ref, out_ref):
        ssum = jnp.sum(part_ref[:, 0, :], axis=0, keepdims=True)
        ssq = jnp.sum(part_ref[:, 1, :], axis=0, keepdims=True)
        m = ssum / e
        v = ssq / e - m * m
        cs = gp_ref[...] * lax.rsqrt(v + 1e-5)
        cb = bp_ref[...] - m * cs
        z = z_ref[...] * cs
        const = (
            jnp.dot(cb, wb_ref[...], preferred_element_type=jnp.float32)
            + bb_ref[...]
        )
        scores = (
            jnp.dot(z, wb_ref[...], preferred_element_type=jnp.float32) + const
        )
        mx = jnp.max(scores, axis=1, keepdims=True)
        ex = jnp.exp(scores - mx)
        lse = jnp.log(jnp.sum(ex, axis=1, keepdims=True)) + mx
        out_ref[...] = scores - lse

    out = pl.pallas_call(
        tc4_body,
        grid=grid,
        in_specs=[
            pl.BlockSpec((NW, 2, h), lambda i: (0, 0, 0)),
            pl.BlockSpec((1, h), lambda i: (0, 0)),
            pl.BlockSpec((1, h), lambda i: (0, 0)),
            pl.BlockSpec((h, cdim), lambda i: (0, 0)),
            pl.BlockSpec((1, cdim), lambda i: (0, 0)),
            pl.BlockSpec((be_blk, h), lambda i: (i, 0)),
        ],
        out_specs=pl.BlockSpec((be_blk, cdim), lambda i: (i, 0)),
        out_shape=jax.ShapeDtypeStruct((e, cdim), jnp.float32),
    )(parts, gp.reshape(1, h), bp.reshape(1, h), Wb, bb.reshape(1, cdim), z_edges)

    return out
